# Initial kernel scaffold; baseline (speedup 1.0000x reference)
import jax
import jax.numpy as jnp
from jax.experimental import pallas as pl


def kernel(x, edge_index, W1, b1, W2, b2):
    raise NotImplementedError("write your pallas kernel here")


# speedup vs baseline: 13.8180x; 13.8180x over previous
import functools

import jax
import jax.numpy as jnp
from jax import lax
from jax.experimental import pallas as pl
from jax.experimental.pallas import tpu as pltpu
from jax.experimental.pallas import tpu_sc as plsc

N_NODES = 10000
NP = 10240
D = 128
D3 = 128
D_OUT = 40
NE = 320000
CHUNK = 128
NC, NS = 2, 16
NW = NC * NS
NCH = 2560
CPW = NCH // NW
EPW = CPW * CHUNK
ROWS_PW = NP // NS
HBINS = NP // 16

_MESH = dict(core_axis_name="c", subcore_axis_name="s", num_cores=NC,
             num_subcores=NS)


def _wid():
    return lax.axis_index("s") * NC + lax.axis_index("c")


def _deg_body(src_hbm, dst_hbm, ones_hbm, zero_hbm, out_hbm,
              sidx_v, didx_v, ones_v, accs_sh, accd_sh):
    c = lax.axis_index("c")
    s = lax.axis_index("s")
    w = s * NC + c
    pltpu.sync_copy(src_hbm.at[pl.ds(w * CPW, CPW)], sidx_v)
    pltpu.sync_copy(dst_hbm.at[pl.ds(w * CPW, CPW)], didx_v)
    pltpu.sync_copy(ones_hbm, ones_v)
    r0 = s * ROWS_PW
    pltpu.sync_copy(zero_hbm.at[pl.ds(r0, ROWS_PW)], accs_sh.at[pl.ds(r0, ROWS_PW)])
    pltpu.sync_copy(zero_hbm.at[pl.ds(r0, ROWS_PW)], accd_sh.at[pl.ds(r0, ROWS_PW)])
    plsc.subcore_barrier()

    def step(j, _):
        pltpu.sync_copy(ones_v, accs_sh.at[sidx_v.at[j]], add=True)
        pltpu.sync_copy(ones_v, accd_sh.at[didx_v.at[j]], add=True)
        return _

    lax.fori_loop(0, CPW, step, 0)
    plsc.subcore_barrier()
    pltpu.sync_copy(accs_sh.at[pl.ds(r0, ROWS_PW)], out_hbm.at[c, 0, pl.ds(r0, ROWS_PW)])
    pltpu.sync_copy(accd_sh.at[pl.ds(r0, ROWS_PW)], out_hbm.at[c, 1, pl.ds(r0, ROWS_PW)])


def _sc_degrees(src2d, dst2d, ones_rows, zero_rows):
    k = pl.kernel(
        _deg_body,
        out_type=jax.ShapeDtypeStruct((NC, 2, NP, 16), jnp.float32),
        mesh=plsc.VectorSubcoreMesh(**_MESH),
        compiler_params=pltpu.CompilerParams(use_tc_tiling_on_sc=False),
        scratch_types=[
            pltpu.VMEM((CPW, CHUNK), jnp.int32),
            pltpu.VMEM((CPW, CHUNK), jnp.int32),
            pltpu.VMEM((CHUNK, 16), jnp.float32),
            pltpu.VMEM_SHARED((NP, 16), jnp.float32),
            pltpu.VMEM_SHARED((NP, 16), jnp.float32),
        ],
    )
    return k(src2d, dst2d, ones_rows, zero_rows)


def _prop_body(table_hbm, src_hbm, dst_hbm, zero_hbm, out_hbm,
               sidx_v, didx_v, buf_v, acc_sh, sem):
    c = lax.axis_index("c")
    s = lax.axis_index("s")
    w = s * NC + c
    pltpu.sync_copy(src_hbm.at[pl.ds(w * CPW, CPW)], sidx_v)
    pltpu.sync_copy(dst_hbm.at[pl.ds(w * CPW, CPW)], didx_v)
    r0 = s * ROWS_PW
    pltpu.sync_copy(zero_hbm.at[pl.ds(r0, ROWS_PW)], acc_sh.at[pl.ds(r0, ROWS_PW)])
    plsc.subcore_barrier()

    def step(j, _):
        pltpu.async_copy(table_hbm.at[sidx_v.at[j]], buf_v, sem).wait()
        pltpu.sync_copy(buf_v, acc_sh.at[didx_v.at[j]], add=True)
        return _

    lax.fori_loop(0, CPW, step, 0)
    plsc.subcore_barrier()
    pltpu.sync_copy(acc_sh.at[pl.ds(r0, ROWS_PW)], out_hbm.at[c, pl.ds(r0, ROWS_PW)])


def _sc_prop(table, src2d, dst2d, zero_rows, d):
    k = pl.kernel(
        _prop_body,
        out_type=jax.ShapeDtypeStruct((NC, NP, d), jnp.float32),
        mesh=plsc.VectorSubcoreMesh(**_MESH),
        scratch_types=[
            pltpu.VMEM((CPW, CHUNK), jnp.int32),
            pltpu.VMEM((CPW, CHUNK), jnp.int32),
            pltpu.VMEM((CHUNK, d), jnp.float32),
            pltpu.VMEM_SHARED((NP, d), jnp.float32),
            pltpu.SemaphoreType.DMA,
        ],
    )
    return k(table, src2d, dst2d, zero_rows)


def _fg_body(dop_ref, dip_ref, f_ref, g_ref):
    deg_o = jnp.sum(dop_ref[...], axis=0)
    deg_i = jnp.sum(dip_ref[...], axis=0)
    rid = lax.broadcasted_iota(jnp.int32, (HBINS, 16), 0)
    cid = lax.broadcasted_iota(jnp.int32, (HBINS, 16), 1)
    valid = (rid * 16 + cid) < N_NODES
    f_ref[...] = jnp.where(valid, lax.rsqrt(jnp.maximum(deg_o, 1.0)), 0.0)
    g_ref[...] = jnp.where(valid, lax.rsqrt(jnp.maximum(deg_i, 1.0)), 0.0)


def _tc_fg(degp):
    return pl.pallas_call(
        _fg_body,
        out_shape=(jax.ShapeDtypeStruct((HBINS, 16), jnp.float32),
                   jax.ShapeDtypeStruct((HBINS, 16), jnp.float32)),
    )(degp[:, 0, :, 0].reshape(NC, HBINS, 16),
      degp[:, 1, :, 0].reshape(NC, HBINS, 16))


def _scale_body(x_ref, f_ref, o_ref):
    o_ref[...] = x_ref[...] * f_ref[...]


def _tc_scale(x, f_col):
    n, d = x.shape
    blk = 1280
    return pl.pallas_call(
        _scale_body,
        grid=(n // blk,),
        in_specs=[pl.BlockSpec((blk, d), lambda i: (i, 0)),
                  pl.BlockSpec((blk, 1), lambda i: (i, 0))],
        out_specs=pl.BlockSpec((blk, d), lambda i: (i, 0)),
        out_shape=jax.ShapeDtypeStruct((n, d), jnp.float32),
    )(x, f_col)


def _lin_body(relu, q0_ref, q1_ref, g_ref, f_ref, w_ref, b_ref, o_ref):
    q = (q0_ref[...] + q1_ref[...]) * g_ref[...]
    h = jnp.dot(q, w_ref[...], preferred_element_type=jnp.float32) + b_ref[...]
    if relu:
        h = jnp.maximum(h, 0.0)
    o_ref[...] = h * f_ref[...]


def _tc_linear(q0, q1, g_col, f_col, w, b_row, relu):
    n, d_in = q0.shape
    d_out = w.shape[1]
    blk = 1280
    return pl.pallas_call(
        functools.partial(_lin_body, relu),
        grid=(n // blk,),
        in_specs=[pl.BlockSpec((blk, d_in), lambda i: (i, 0)),
                  pl.BlockSpec((blk, d_in), lambda i: (i, 0)),
                  pl.BlockSpec((blk, 1), lambda i: (i, 0)),
                  pl.BlockSpec((blk, 1), lambda i: (i, 0)),
                  pl.BlockSpec((d_in, d_out), lambda i: (0, 0)),
                  pl.BlockSpec((1, d_out), lambda i: (0, 0))],
        out_specs=pl.BlockSpec((blk, d_out), lambda i: (i, 0)),
        out_shape=jax.ShapeDtypeStruct((n, d_out), jnp.float32),
    )(q0, q1, g_col, f_col, w, b_row)


def _final_body(t0_ref, t1_ref, g_ref, ls_ref, h_ref):
    h = (t0_ref[...] + t1_ref[...]) * g_ref[...]
    col = lax.broadcasted_iota(jnp.int32, h.shape, 1)
    mask = col < D_OUT
    neg = jnp.float32(-1e30)
    m = jnp.max(jnp.where(mask, h, neg), axis=1, keepdims=True)
    e = jnp.where(mask, jnp.exp(h - m), 0.0)
    lse = jnp.log(jnp.sum(e, axis=1, keepdims=True)) + m
    h_ref[...] = h
    ls_ref[...] = h - lse


def _tc_final(t0, t1, g_col):
    n, d = t0.shape
    blk = 1280
    return pl.pallas_call(
        _final_body,
        grid=(n // blk,),
        in_specs=[pl.BlockSpec((blk, d), lambda i: (i, 0)),
                  pl.BlockSpec((blk, d), lambda i: (i, 0)),
                  pl.BlockSpec((blk, 1), lambda i: (i, 0))],
        out_specs=(pl.BlockSpec((blk, d), lambda i: (i, 0)),
                   pl.BlockSpec((blk, d), lambda i: (i, 0))),
        out_shape=(jax.ShapeDtypeStruct((n, d), jnp.float32),
                   jax.ShapeDtypeStruct((n, d), jnp.float32)),
    )(t0, t1, g_col)


def kernel(x, edge_index, W1, b1, W2, b2):
    src = edge_index[0].astype(jnp.int32)
    dst = edge_index[1].astype(jnp.int32)
    n_pad = NCH * CHUNK - NE
    pad_idx = N_NODES + (jnp.arange(n_pad, dtype=jnp.int32) % (NP - N_NODES))
    src_flat = jnp.concatenate([src, pad_idx])
    dst_flat = jnp.concatenate([dst, pad_idx])
    src2d = src_flat.reshape(NCH, CHUNK)
    dst2d = dst_flat.reshape(NCH, CHUNK)

    x_pad = jnp.pad(x, ((0, NP - N_NODES), (0, 0)))
    w2p = jnp.pad(W2, ((0, 0), (0, D3 - D_OUT)))
    b1r = b1.reshape(1, D)
    b2r = jnp.pad(b2, (0, D3 - D_OUT)).reshape(1, D3)
    zeros_d = jnp.zeros((NP, D), jnp.float32)
    zeros_16 = jnp.zeros((NP, 16), jnp.float32)
    ones_rows = jnp.ones((CHUNK, 16), jnp.float32)

    degp = _sc_degrees(src2d, dst2d, ones_rows, zeros_16)
    f2d, g2d = _tc_fg(degp)
    f_col = f2d.reshape(NP, 1)
    g_col = g2d.reshape(NP, 1)

    xs = _tc_scale(x_pad, f_col)
    q = _sc_prop(xs, src2d, dst2d, zeros_d, D)
    h1s = _tc_linear(q[0], q[1], g_col, f_col, W1, b1r, relu=True)
    r = _sc_prop(h1s, src2d, dst2d, zeros_d, D)
    h2s = _tc_linear(r[0], r[1], g_col, f_col, w2p, b2r, relu=False)
    t = _sc_prop(h2s, src2d, dst2d, zeros_d, D3)
    ls, h = _tc_final(t[0], t[1], g_col)
    return (ls[:N_NODES, :D_OUT], h[:N_NODES, :D_OUT])


# speedup vs baseline: 16.9948x; 1.2299x over previous
import functools

import jax
import jax.numpy as jnp
from jax import lax
from jax.experimental import pallas as pl
from jax.experimental.pallas import tpu as pltpu
from jax.experimental.pallas import tpu_sc as plsc

N_NODES = 10000
NP = 10240
D = 128
D3 = 128
D_OUT = 40
NE = 320000
CHUNK = 128
NC, NS = 2, 16
NW = NC * NS
NCH = 2560
CPW = NCH // NW
EPW = CPW * CHUNK
ROWS_PW = NP // NS
HBINS = NP // 16

_MESH = dict(core_axis_name="c", subcore_axis_name="s", num_cores=NC,
             num_subcores=NS)


def _wid():
    return lax.axis_index("s") * NC + lax.axis_index("c")


def _deg_body(src_hbm, dst_hbm, ones_hbm, zero_hbm, out_hbm,
              sidx_v, didx_v, ones_v, accs_sh, accd_sh):
    c = lax.axis_index("c")
    s = lax.axis_index("s")
    w = s * NC + c
    pltpu.sync_copy(src_hbm.at[pl.ds(w * CPW, CPW)], sidx_v)
    pltpu.sync_copy(dst_hbm.at[pl.ds(w * CPW, CPW)], didx_v)
    pltpu.sync_copy(ones_hbm, ones_v)
    r0 = s * ROWS_PW
    pltpu.sync_copy(zero_hbm.at[pl.ds(r0, ROWS_PW)], accs_sh.at[pl.ds(r0, ROWS_PW)])
    pltpu.sync_copy(zero_hbm.at[pl.ds(r0, ROWS_PW)], accd_sh.at[pl.ds(r0, ROWS_PW)])
    plsc.subcore_barrier()

    def step(j, _):
        pltpu.sync_copy(ones_v, accs_sh.at[sidx_v.at[j]], add=True)
        pltpu.sync_copy(ones_v, accd_sh.at[didx_v.at[j]], add=True)
        return _

    lax.fori_loop(0, CPW, step, 0)
    plsc.subcore_barrier()
    pltpu.sync_copy(accs_sh.at[pl.ds(r0, ROWS_PW)], out_hbm.at[c, 0, pl.ds(r0, ROWS_PW)])
    pltpu.sync_copy(accd_sh.at[pl.ds(r0, ROWS_PW)], out_hbm.at[c, 1, pl.ds(r0, ROWS_PW)])


def _sc_degrees(src2d, dst2d, ones_rows, zero_rows):
    k = pl.kernel(
        _deg_body,
        out_type=jax.ShapeDtypeStruct((NC, 2, NP, 16), jnp.float32),
        mesh=plsc.VectorSubcoreMesh(**_MESH),
        compiler_params=pltpu.CompilerParams(use_tc_tiling_on_sc=False),
        scratch_types=[
            pltpu.VMEM((CPW, CHUNK), jnp.int32),
            pltpu.VMEM((CPW, CHUNK), jnp.int32),
            pltpu.VMEM((CHUNK, 16), jnp.float32),
            pltpu.VMEM_SHARED((NP, 16), jnp.float32),
            pltpu.VMEM_SHARED((NP, 16), jnp.float32),
        ],
    )
    return k(src2d, dst2d, ones_rows, zero_rows)


NBUF = 2


HCH = CPW // 2


def _prop_body(table_hbm, src_hbm, dst_hbm, zero_hbm, out_hbm,
               sidx_v, didx_v, buf0_v, buf1_v, acc_sh, gsem0, gsem1):
    c = lax.axis_index("c")
    s = lax.axis_index("s")
    w = s * NC + c
    bufs = (buf0_v, buf1_v)
    gsems = (gsem0, gsem1)
    r0 = s * ROWS_PW
    pltpu.sync_copy(zero_hbm.at[pl.ds(r0, ROWS_PW)], acc_sh.at[pl.ds(r0, ROWS_PW)])
    plsc.subcore_barrier()

    for h in range(2):
        base = w * CPW + h * HCH
        pltpu.sync_copy(src_hbm.at[pl.ds(base, HCH)], sidx_v)
        pltpu.sync_copy(dst_hbm.at[pl.ds(base, HCH)], didx_v)
        pltpu.async_copy(table_hbm.at[sidx_v.at[0]], bufs[0], gsems[0])

        def group(g, _):
            for b in range(NBUF):
                j = g * NBUF + b
                pltpu.make_async_copy(
                    table_hbm.at[sidx_v.at[j]], bufs[b], gsems[b]).wait()

                @pl.when(j + 1 < HCH)
                def _issue():
                    pltpu.async_copy(table_hbm.at[sidx_v.at[j + 1]],
                                     bufs[1 - b], gsems[1 - b])

                pltpu.sync_copy(bufs[b], acc_sh.at[didx_v.at[j]], add=True)
            return _

        lax.fori_loop(0, HCH // NBUF, group, 0)
    plsc.subcore_barrier()
    pltpu.sync_copy(acc_sh.at[pl.ds(r0, ROWS_PW)], out_hbm.at[c, pl.ds(r0, ROWS_PW)])


def _sc_prop(table, src2d, dst2d, zero_rows, d):
    k = pl.kernel(
        _prop_body,
        out_type=jax.ShapeDtypeStruct((NC, NP, d), jnp.float32),
        mesh=plsc.VectorSubcoreMesh(**_MESH),
        scratch_types=[
            pltpu.VMEM((HCH, CHUNK), jnp.int32),
            pltpu.VMEM((HCH, CHUNK), jnp.int32),
            pltpu.VMEM((CHUNK, d), jnp.float32),
            pltpu.VMEM((CHUNK, d), jnp.float32),
            pltpu.VMEM_SHARED((NP, d), jnp.float32),
            pltpu.SemaphoreType.DMA,
            pltpu.SemaphoreType.DMA,
        ],
    )
    return k(table, src2d, dst2d, zero_rows)


def _fg_body(dop_ref, dip_ref, f_ref, g_ref):
    deg_o = jnp.sum(dop_ref[...], axis=0)
    deg_i = jnp.sum(dip_ref[...], axis=0)
    rid = lax.broadcasted_iota(jnp.int32, (HBINS, 16), 0)
    cid = lax.broadcasted_iota(jnp.int32, (HBINS, 16), 1)
    valid = (rid * 16 + cid) < N_NODES
    f_ref[...] = jnp.where(valid, lax.rsqrt(jnp.maximum(deg_o, 1.0)), 0.0)
    g_ref[...] = jnp.where(valid, lax.rsqrt(jnp.maximum(deg_i, 1.0)), 0.0)


def _tc_fg(degp):
    return pl.pallas_call(
        _fg_body,
        out_shape=(jax.ShapeDtypeStruct((HBINS, 16), jnp.float32),
                   jax.ShapeDtypeStruct((HBINS, 16), jnp.float32)),
    )(degp[:, 0, :, 0].reshape(NC, HBINS, 16),
      degp[:, 1, :, 0].reshape(NC, HBINS, 16))


def _scale_body(x_ref, f_ref, o_ref):
    o_ref[...] = x_ref[...] * f_ref[...]


def _tc_scale(x, f_col):
    n, d = x.shape
    blk = 1280
    return pl.pallas_call(
        _scale_body,
        grid=(n // blk,),
        in_specs=[pl.BlockSpec((blk, d), lambda i: (i, 0)),
                  pl.BlockSpec((blk, 1), lambda i: (i, 0))],
        out_specs=pl.BlockSpec((blk, d), lambda i: (i, 0)),
        out_shape=jax.ShapeDtypeStruct((n, d), jnp.float32),
    )(x, f_col)


def _lin_body(relu, q0_ref, q1_ref, g_ref, f_ref, w_ref, b_ref, o_ref):
    q = (q0_ref[...] + q1_ref[...]) * g_ref[...]
    h = jnp.dot(q, w_ref[...], preferred_element_type=jnp.float32) + b_ref[...]
    if relu:
        h = jnp.maximum(h, 0.0)
    o_ref[...] = h * f_ref[...]


def _tc_linear(q0, q1, g_col, f_col, w, b_row, relu):
    n, d_in = q0.shape
    d_out = w.shape[1]
    blk = 1280
    return pl.pallas_call(
        functools.partial(_lin_body, relu),
        grid=(n // blk,),
        in_specs=[pl.BlockSpec((blk, d_in), lambda i: (i, 0)),
                  pl.BlockSpec((blk, d_in), lambda i: (i, 0)),
                  pl.BlockSpec((blk, 1), lambda i: (i, 0)),
                  pl.BlockSpec((blk, 1), lambda i: (i, 0)),
                  pl.BlockSpec((d_in, d_out), lambda i: (0, 0)),
                  pl.BlockSpec((1, d_out), lambda i: (0, 0))],
        out_specs=pl.BlockSpec((blk, d_out), lambda i: (i, 0)),
        out_shape=jax.ShapeDtypeStruct((n, d_out), jnp.float32),
    )(q0, q1, g_col, f_col, w, b_row)


def _final_body(t0_ref, t1_ref, g_ref, ls_ref, h_ref):
    h = (t0_ref[...] + t1_ref[...]) * g_ref[...]
    col = lax.broadcasted_iota(jnp.int32, h.shape, 1)
    mask = col < D_OUT
    neg = jnp.float32(-1e30)
    m = jnp.max(jnp.where(mask, h, neg), axis=1, keepdims=True)
    e = jnp.where(mask, jnp.exp(h - m), 0.0)
    lse = jnp.log(jnp.sum(e, axis=1, keepdims=True)) + m
    h_ref[...] = h
    ls_ref[...] = h - lse


def _tc_final(t0, t1, g_col):
    n, d = t0.shape
    blk = 1280
    return pl.pallas_call(
        _final_body,
        grid=(n // blk,),
        in_specs=[pl.BlockSpec((blk, d), lambda i: (i, 0)),
                  pl.BlockSpec((blk, d), lambda i: (i, 0)),
                  pl.BlockSpec((blk, 1), lambda i: (i, 0))],
        out_specs=(pl.BlockSpec((blk, d), lambda i: (i, 0)),
                   pl.BlockSpec((blk, d), lambda i: (i, 0))),
        out_shape=(jax.ShapeDtypeStruct((n, d), jnp.float32),
                   jax.ShapeDtypeStruct((n, d), jnp.float32)),
    )(t0, t1, g_col)


def kernel(x, edge_index, W1, b1, W2, b2):
    src = edge_index[0].astype(jnp.int32)
    dst = edge_index[1].astype(jnp.int32)
    n_pad = NCH * CHUNK - NE
    pad_idx = N_NODES + (jnp.arange(n_pad, dtype=jnp.int32) % (NP - N_NODES))
    src_flat = jnp.concatenate([src, pad_idx])
    dst_flat = jnp.concatenate([dst, pad_idx])
    src2d = src_flat.reshape(NCH, CHUNK)
    dst2d = dst_flat.reshape(NCH, CHUNK)

    x_pad = jnp.pad(x, ((0, NP - N_NODES), (0, 0)))
    w2p = jnp.pad(W2, ((0, 0), (0, D3 - D_OUT)))
    b1r = b1.reshape(1, D)
    b2r = jnp.pad(b2, (0, D3 - D_OUT)).reshape(1, D3)
    zeros_d = jnp.zeros((NP, D), jnp.float32)
    zeros_16 = jnp.zeros((NP, 16), jnp.float32)
    ones_rows = jnp.ones((CHUNK, 16), jnp.float32)

    degp = _sc_degrees(src2d, dst2d, ones_rows, zeros_16)
    f2d, g2d = _tc_fg(degp)
    f_col = f2d.reshape(NP, 1)
    g_col = g2d.reshape(NP, 1)

    xs = _tc_scale(x_pad, f_col)
    q = _sc_prop(xs, src2d, dst2d, zeros_d, D)
    h1s = _tc_linear(q[0], q[1], g_col, f_col, W1, b1r, relu=True)
    r = _sc_prop(h1s, src2d, dst2d, zeros_d, D)
    h2s = _tc_linear(r[0], r[1], g_col, f_col, w2p, b2r, relu=False)
    t = _sc_prop(h2s, src2d, dst2d, zeros_d, D3)
    ls, h = _tc_final(t[0], t[1], g_col)
    return (ls[:N_NODES, :D_OUT], h[:N_NODES, :D_OUT])


# speedup vs baseline: 17.8895x; 1.0526x over previous
import functools

import jax
import jax.numpy as jnp
from jax import lax
from jax.experimental import pallas as pl
from jax.experimental.pallas import tpu as pltpu
from jax.experimental.pallas import tpu_sc as plsc

N_NODES = 10000
NP = 10240
D = 128
D3 = 64
D_OUT = 40
NE = 320000
CHUNK = 128
NC, NS = 2, 16
NW = NC * NS
NCH = 2560
CPW = NCH // NW
EPW = CPW * CHUNK
ROWS_PW = NP // NS
HBINS = NP // 16

_MESH = dict(core_axis_name="c", subcore_axis_name="s", num_cores=NC,
             num_subcores=NS)


def _wid():
    return lax.axis_index("s") * NC + lax.axis_index("c")


def _deg_body(src_hbm, dst_hbm, ones_hbm, zero_hbm, out_hbm,
              sidx_v, didx_v, ones_v, accs_sh, accd_sh):
    c = lax.axis_index("c")
    s = lax.axis_index("s")
    w = s * NC + c
    pltpu.sync_copy(src_hbm.at[pl.ds(w * CPW, CPW)], sidx_v)
    pltpu.sync_copy(dst_hbm.at[pl.ds(w * CPW, CPW)], didx_v)
    pltpu.sync_copy(ones_hbm, ones_v)
    r0 = s * ROWS_PW
    pltpu.sync_copy(zero_hbm.at[pl.ds(r0, ROWS_PW)], accs_sh.at[pl.ds(r0, ROWS_PW)])
    pltpu.sync_copy(zero_hbm.at[pl.ds(r0, ROWS_PW)], accd_sh.at[pl.ds(r0, ROWS_PW)])
    plsc.subcore_barrier()

    def step(j, _):
        pltpu.sync_copy(ones_v, accs_sh.at[sidx_v.at[j]], add=True)
        pltpu.sync_copy(ones_v, accd_sh.at[didx_v.at[j]], add=True)
        return _

    lax.fori_loop(0, CPW, step, 0)
    plsc.subcore_barrier()
    pltpu.sync_copy(accs_sh.at[pl.ds(r0, ROWS_PW)], out_hbm.at[c, 0, pl.ds(r0, ROWS_PW)])
    pltpu.sync_copy(accd_sh.at[pl.ds(r0, ROWS_PW)], out_hbm.at[c, 1, pl.ds(r0, ROWS_PW)])


def _sc_degrees(src2d, dst2d, ones_rows, zero_rows):
    k = pl.kernel(
        _deg_body,
        out_type=jax.ShapeDtypeStruct((NC, 2, NP, 16), jnp.float32),
        mesh=plsc.VectorSubcoreMesh(**_MESH),
        compiler_params=pltpu.CompilerParams(use_tc_tiling_on_sc=False),
        scratch_types=[
            pltpu.VMEM((CPW, CHUNK), jnp.int32),
            pltpu.VMEM((CPW, CHUNK), jnp.int32),
            pltpu.VMEM((CHUNK, 16), jnp.float32),
            pltpu.VMEM_SHARED((NP, 16), jnp.float32),
            pltpu.VMEM_SHARED((NP, 16), jnp.float32),
        ],
    )
    return k(src2d, dst2d, ones_rows, zero_rows)


NBUF = 2


HCH = CPW // 2


def _prop_body(table_hbm, src_hbm, dst_hbm, zero_hbm, out_hbm,
               sidx_v, didx_v, buf0_v, buf1_v, acc_sh, gsem0, gsem1):
    c = lax.axis_index("c")
    s = lax.axis_index("s")
    w = s * NC + c
    bufs = (buf0_v, buf1_v)
    gsems = (gsem0, gsem1)
    r0 = s * ROWS_PW
    pltpu.sync_copy(zero_hbm.at[pl.ds(r0, ROWS_PW)], acc_sh.at[pl.ds(r0, ROWS_PW)])
    plsc.subcore_barrier()

    for h in range(2):
        base = w * CPW + h * HCH
        pltpu.sync_copy(src_hbm.at[pl.ds(base, HCH)], sidx_v)
        pltpu.sync_copy(dst_hbm.at[pl.ds(base, HCH)], didx_v)
        pltpu.async_copy(table_hbm.at[sidx_v.at[0]], bufs[0], gsems[0])

        def group(g, _):
            for b in range(NBUF):
                j = g * NBUF + b
                pltpu.make_async_copy(
                    table_hbm.at[sidx_v.at[j]], bufs[b], gsems[b]).wait()

                @pl.when(j + 1 < HCH)
                def _issue():
                    pltpu.async_copy(table_hbm.at[sidx_v.at[j + 1]],
                                     bufs[1 - b], gsems[1 - b])

                pltpu.sync_copy(bufs[b], acc_sh.at[didx_v.at[j]], add=True)
            return _

        lax.fori_loop(0, HCH // NBUF, group, 0)
    plsc.subcore_barrier()
    pltpu.sync_copy(acc_sh.at[pl.ds(r0, ROWS_PW)], out_hbm.at[c, pl.ds(r0, ROWS_PW)])


def _sc_prop(table, src2d, dst2d, zero_rows, d):
    params = (pltpu.CompilerParams(use_tc_tiling_on_sc=False)
              if d < 128 else None)
    k = pl.kernel(
        _prop_body,
        out_type=jax.ShapeDtypeStruct((NC, NP, d), jnp.float32),
        mesh=plsc.VectorSubcoreMesh(**_MESH),
        compiler_params=params,
        scratch_types=[
            pltpu.VMEM((HCH, CHUNK), jnp.int32),
            pltpu.VMEM((HCH, CHUNK), jnp.int32),
            pltpu.VMEM((CHUNK, d), jnp.float32),
            pltpu.VMEM((CHUNK, d), jnp.float32),
            pltpu.VMEM_SHARED((NP, d), jnp.float32),
            pltpu.SemaphoreType.DMA,
            pltpu.SemaphoreType.DMA,
        ],
    )
    return k(table, src2d, dst2d, zero_rows)


def _fg_body(dop_ref, dip_ref, f_ref, g_ref):
    deg_o = jnp.sum(dop_ref[...], axis=0)
    deg_i = jnp.sum(dip_ref[...], axis=0)
    rid = lax.broadcasted_iota(jnp.int32, (HBINS, 16), 0)
    cid = lax.broadcasted_iota(jnp.int32, (HBINS, 16), 1)
    valid = (rid * 16 + cid) < N_NODES
    f_ref[...] = jnp.where(valid, lax.rsqrt(jnp.maximum(deg_o, 1.0)), 0.0)
    g_ref[...] = jnp.where(valid, lax.rsqrt(jnp.maximum(deg_i, 1.0)), 0.0)


def _tc_fg(degp):
    return pl.pallas_call(
        _fg_body,
        out_shape=(jax.ShapeDtypeStruct((HBINS, 16), jnp.float32),
                   jax.ShapeDtypeStruct((HBINS, 16), jnp.float32)),
    )(degp[:, 0, :, 0].reshape(NC, HBINS, 16),
      degp[:, 1, :, 0].reshape(NC, HBINS, 16))


def _scale_body(x_ref, f_ref, o_ref):
    o_ref[...] = x_ref[...] * f_ref[...]


def _tc_scale(x, f_col):
    n, d = x.shape
    blk = 1280
    return pl.pallas_call(
        _scale_body,
        grid=(n // blk,),
        in_specs=[pl.BlockSpec((blk, d), lambda i: (i, 0)),
                  pl.BlockSpec((blk, 1), lambda i: (i, 0))],
        out_specs=pl.BlockSpec((blk, d), lambda i: (i, 0)),
        out_shape=jax.ShapeDtypeStruct((n, d), jnp.float32),
    )(x, f_col)


def _lin_body(relu, q0_ref, q1_ref, g_ref, f_ref, w_ref, b_ref, o_ref):
    q = (q0_ref[...] + q1_ref[...]) * g_ref[...]
    h = jnp.dot(q, w_ref[...], preferred_element_type=jnp.float32) + b_ref[...]
    if relu:
        h = jnp.maximum(h, 0.0)
    o_ref[...] = h * f_ref[...]


def _tc_linear(q0, q1, g_col, f_col, w, b_row, relu):
    n, d_in = q0.shape
    d_out = w.shape[1]
    blk = 1280
    return pl.pallas_call(
        functools.partial(_lin_body, relu),
        grid=(n // blk,),
        in_specs=[pl.BlockSpec((blk, d_in), lambda i: (i, 0)),
                  pl.BlockSpec((blk, d_in), lambda i: (i, 0)),
                  pl.BlockSpec((blk, 1), lambda i: (i, 0)),
                  pl.BlockSpec((blk, 1), lambda i: (i, 0)),
                  pl.BlockSpec((d_in, d_out), lambda i: (0, 0)),
                  pl.BlockSpec((1, d_out), lambda i: (0, 0))],
        out_specs=pl.BlockSpec((blk, d_out), lambda i: (i, 0)),
        out_shape=jax.ShapeDtypeStruct((n, d_out), jnp.float32),
    )(q0, q1, g_col, f_col, w, b_row)


def _final_body(t0_ref, t1_ref, g_ref, ls_ref, h_ref):
    h = (t0_ref[...] + t1_ref[...]) * g_ref[...]
    col = lax.broadcasted_iota(jnp.int32, h.shape, 1)
    mask = col < D_OUT
    neg = jnp.float32(-1e30)
    m = jnp.max(jnp.where(mask, h, neg), axis=1, keepdims=True)
    e = jnp.where(mask, jnp.exp(h - m), 0.0)
    lse = jnp.log(jnp.sum(e, axis=1, keepdims=True)) + m
    h_ref[...] = h
    ls_ref[...] = h - lse


def _tc_final(t0, t1, g_col):
    n, d = t0.shape
    blk = 1280
    return pl.pallas_call(
        _final_body,
        grid=(n // blk,),
        in_specs=[pl.BlockSpec((blk, d), lambda i: (i, 0)),
                  pl.BlockSpec((blk, d), lambda i: (i, 0)),
                  pl.BlockSpec((blk, 1), lambda i: (i, 0))],
        out_specs=(pl.BlockSpec((blk, d), lambda i: (i, 0)),
                   pl.BlockSpec((blk, d), lambda i: (i, 0))),
        out_shape=(jax.ShapeDtypeStruct((n, d), jnp.float32),
                   jax.ShapeDtypeStruct((n, d), jnp.float32)),
    )(t0, t1, g_col)


def kernel(x, edge_index, W1, b1, W2, b2):
    src = edge_index[0].astype(jnp.int32)
    dst = edge_index[1].astype(jnp.int32)
    n_pad = NCH * CHUNK - NE
    pad_idx = N_NODES + (jnp.arange(n_pad, dtype=jnp.int32) % (NP - N_NODES))
    src_flat = jnp.concatenate([src, pad_idx])
    dst_flat = jnp.concatenate([dst, pad_idx])
    src2d = src_flat.reshape(NCH, CHUNK)
    dst2d = dst_flat.reshape(NCH, CHUNK)

    x_pad = jnp.pad(x, ((0, NP - N_NODES), (0, 0)))
    w2p = jnp.pad(W2, ((0, 0), (0, D3 - D_OUT)))
    b1r = b1.reshape(1, D)
    b2r = jnp.pad(b2, (0, D3 - D_OUT)).reshape(1, D3)
    zeros_d = jnp.zeros((NP, D), jnp.float32)
    zeros_d3 = jnp.zeros((NP, D3), jnp.float32)
    zeros_16 = jnp.zeros((NP, 16), jnp.float32)
    ones_rows = jnp.ones((CHUNK, 16), jnp.float32)

    degp = _sc_degrees(src2d, dst2d, ones_rows, zeros_16)
    f2d, g2d = _tc_fg(degp)
    f_col = f2d.reshape(NP, 1)
    g_col = g2d.reshape(NP, 1)

    xs = _tc_scale(x_pad, f_col)
    q = _sc_prop(xs, src2d, dst2d, zeros_d, D)
    h1s = _tc_linear(q[0], q[1], g_col, f_col, W1, b1r, relu=True)
    r = _sc_prop(h1s, src2d, dst2d, zeros_d, D)
    h2s = _tc_linear(r[0], r[1], g_col, f_col, w2p, b2r, relu=False)
    t = _sc_prop(h2s, src2d, dst2d, zeros_d3, D3)
    ls, h = _tc_final(t[0], t[1], g_col)
    return (ls[:N_NODES, :D_OUT], h[:N_NODES, :D_OUT])


# speedup vs baseline: 18.2972x; 1.0228x over previous
import functools

import jax
import jax.numpy as jnp
from jax import lax
from jax.experimental import pallas as pl
from jax.experimental.pallas import tpu as pltpu
from jax.experimental.pallas import tpu_sc as plsc

N_NODES = 10000
NP = 10240
D = 128
D3 = 64
D_OUT = 40
NE = 320000
CHUNK = 128
NC, NS = 2, 16
NW = NC * NS
NCH = 2560
CPW = NCH // NW
EPW = CPW * CHUNK
ROWS_PW = NP // NS
HBINS = NP // 16

_MESH = dict(core_axis_name="c", subcore_axis_name="s", num_cores=NC,
             num_subcores=NS)


def _wid():
    return lax.axis_index("s") * NC + lax.axis_index("c")


def _deg_body(src_hbm, dst_hbm, ones_hbm, zero_hbm, out_hbm,
              sidx_v, didx_v, ones_v, accs_sh, accd_sh):
    c = lax.axis_index("c")
    s = lax.axis_index("s")
    w = s * NC + c
    pltpu.sync_copy(src_hbm.at[pl.ds(w * CPW, CPW)], sidx_v)
    pltpu.sync_copy(dst_hbm.at[pl.ds(w * CPW, CPW)], didx_v)
    pltpu.sync_copy(ones_hbm, ones_v)
    r0 = s * ROWS_PW
    pltpu.sync_copy(zero_hbm.at[pl.ds(r0, ROWS_PW)], accs_sh.at[pl.ds(r0, ROWS_PW)])
    pltpu.sync_copy(zero_hbm.at[pl.ds(r0, ROWS_PW)], accd_sh.at[pl.ds(r0, ROWS_PW)])
    plsc.subcore_barrier()

    def step(j, _):
        pltpu.sync_copy(ones_v, accs_sh.at[sidx_v.at[j]], add=True)
        pltpu.sync_copy(ones_v, accd_sh.at[didx_v.at[j]], add=True)
        return _

    lax.fori_loop(0, CPW, step, 0)
    plsc.subcore_barrier()
    pltpu.sync_copy(accs_sh.at[pl.ds(r0, ROWS_PW)], out_hbm.at[c, 0, pl.ds(r0, ROWS_PW)])
    pltpu.sync_copy(accd_sh.at[pl.ds(r0, ROWS_PW)], out_hbm.at[c, 1, pl.ds(r0, ROWS_PW)])


def _sc_degrees(src2d, dst2d, ones_rows, zero_rows):
    k = pl.kernel(
        _deg_body,
        out_type=jax.ShapeDtypeStruct((NC, 2, NP, 16), jnp.float32),
        mesh=plsc.VectorSubcoreMesh(**_MESH),
        compiler_params=pltpu.CompilerParams(use_tc_tiling_on_sc=False),
        scratch_types=[
            pltpu.VMEM((CPW, CHUNK), jnp.int32),
            pltpu.VMEM((CPW, CHUNK), jnp.int32),
            pltpu.VMEM((CHUNK, 16), jnp.float32),
            pltpu.VMEM_SHARED((NP, 16), jnp.float32),
            pltpu.VMEM_SHARED((NP, 16), jnp.float32),
        ],
    )
    return k(src2d, dst2d, ones_rows, zero_rows)


NBUF = 2


HCH = CPW // 2


def _prop_body(table_hbm, src_hbm, dst_hbm, zero_hbm, out_hbm,
               sidx_v, didx_v, buf0_v, buf1_v, acc_sh, gsem0, gsem1):
    c = lax.axis_index("c")
    s = lax.axis_index("s")
    w = s * NC + c
    bufs = (buf0_v, buf1_v)
    gsems = (gsem0, gsem1)
    r0 = s * ROWS_PW
    pltpu.sync_copy(zero_hbm.at[pl.ds(r0, ROWS_PW)], acc_sh.at[pl.ds(r0, ROWS_PW)])
    plsc.subcore_barrier()

    for h in range(2):
        base = w * CPW + h * HCH
        pltpu.sync_copy(src_hbm.at[pl.ds(base, HCH)], sidx_v)
        pltpu.sync_copy(dst_hbm.at[pl.ds(base, HCH)], didx_v)
        pltpu.async_copy(table_hbm.at[sidx_v.at[0]], bufs[0], gsems[0])

        def group(g, _):
            for b in range(NBUF):
                j = g * NBUF + b
                pltpu.make_async_copy(
                    table_hbm.at[sidx_v.at[j]], bufs[b], gsems[b]).wait()

                @pl.when(j + 1 < HCH)
                def _issue():
                    pltpu.async_copy(table_hbm.at[sidx_v.at[j + 1]],
                                     bufs[1 - b], gsems[1 - b])

                pltpu.sync_copy(bufs[b], acc_sh.at[didx_v.at[j]], add=True)
            return _

        lax.fori_loop(0, HCH // NBUF, group, 0)
    plsc.subcore_barrier()
    pltpu.sync_copy(acc_sh.at[pl.ds(r0, ROWS_PW)], out_hbm.at[c, pl.ds(r0, ROWS_PW)])


def _sc_prop(table, src2d, dst2d, zero_rows, d):
    params = (pltpu.CompilerParams(use_tc_tiling_on_sc=False)
              if d < 128 else None)
    k = pl.kernel(
        _prop_body,
        out_type=jax.ShapeDtypeStruct((NC, NP, d), jnp.float32),
        mesh=plsc.VectorSubcoreMesh(**_MESH),
        compiler_params=params,
        scratch_types=[
            pltpu.VMEM((HCH, CHUNK), jnp.int32),
            pltpu.VMEM((HCH, CHUNK), jnp.int32),
            pltpu.VMEM((CHUNK, d), jnp.float32),
            pltpu.VMEM((CHUNK, d), jnp.float32),
            pltpu.VMEM_SHARED((NP, d), jnp.float32),
            pltpu.SemaphoreType.DMA,
            pltpu.SemaphoreType.DMA,
        ],
    )
    return k(table, src2d, dst2d, zero_rows)


def _fg_body(degp_ref, f_ref, g_ref):
    deg_o = degp_ref[0, 0, :, 0:1] + degp_ref[1, 0, :, 0:1]
    deg_i = degp_ref[0, 1, :, 0:1] + degp_ref[1, 1, :, 0:1]
    valid = lax.broadcasted_iota(jnp.int32, (NP, 1), 0) < N_NODES
    f_ref[...] = jnp.where(valid, lax.rsqrt(jnp.maximum(deg_o, 1.0)), 0.0)
    g_ref[...] = jnp.where(valid, lax.rsqrt(jnp.maximum(deg_i, 1.0)), 0.0)


def _tc_fg(degp):
    return pl.pallas_call(
        _fg_body,
        out_shape=(jax.ShapeDtypeStruct((NP, 1), jnp.float32),
                   jax.ShapeDtypeStruct((NP, 1), jnp.float32)),
    )(degp)


def _scale_body(x_ref, f_ref, o_ref):
    o_ref[...] = x_ref[...] * f_ref[...]


def _tc_scale(x, f_col):
    d = x.shape[1]
    blk = 2048
    return pl.pallas_call(
        _scale_body,
        grid=(NP // blk,),
        in_specs=[pl.BlockSpec((blk, d), lambda i: (i, 0)),
                  pl.BlockSpec((blk, 1), lambda i: (i, 0))],
        out_specs=pl.BlockSpec((blk, d), lambda i: (i, 0)),
        out_shape=jax.ShapeDtypeStruct((NP, d), jnp.float32),
    )(x, f_col)


def _lin_body(relu, q0_ref, q1_ref, g_ref, f_ref, w_ref, b_ref, o_ref):
    q = (q0_ref[...] + q1_ref[...]) * g_ref[...]
    h = jnp.dot(q, w_ref[...], preferred_element_type=jnp.float32) + b_ref[...]
    if relu:
        h = jnp.maximum(h, 0.0)
    o_ref[...] = h * f_ref[...]


def _tc_linear(q0, q1, g_col, f_col, w, b_row, relu):
    n, d_in = q0.shape
    d_out = w.shape[1]
    blk = 2048
    return pl.pallas_call(
        functools.partial(_lin_body, relu),
        grid=(n // blk,),
        in_specs=[pl.BlockSpec((blk, d_in), lambda i: (i, 0)),
                  pl.BlockSpec((blk, d_in), lambda i: (i, 0)),
                  pl.BlockSpec((blk, 1), lambda i: (i, 0)),
                  pl.BlockSpec((blk, 1), lambda i: (i, 0)),
                  pl.BlockSpec((d_in, d_out), lambda i: (0, 0)),
                  pl.BlockSpec((1, d_out), lambda i: (0, 0))],
        out_specs=pl.BlockSpec((blk, d_out), lambda i: (i, 0)),
        out_shape=jax.ShapeDtypeStruct((n, d_out), jnp.float32),
    )(q0, q1, g_col, f_col, w, b_row)


def _final_body(t0_ref, t1_ref, g_ref, ls_ref, h_ref):
    h = ((t0_ref[...] + t1_ref[...]) * g_ref[...])[:, :D_OUT]
    m = jnp.max(h, axis=1, keepdims=True)
    e = jnp.exp(h - m)
    lse = jnp.log(jnp.sum(e, axis=1, keepdims=True)) + m
    h_ref[...] = h
    ls_ref[...] = h - lse


def _tc_final(t0, t1, g_col):
    d = t0.shape[1]
    blk = 1000
    return pl.pallas_call(
        _final_body,
        grid=(N_NODES // blk,),
        in_specs=[pl.BlockSpec((blk, d), lambda i: (i, 0)),
                  pl.BlockSpec((blk, d), lambda i: (i, 0)),
                  pl.BlockSpec((blk, 1), lambda i: (i, 0))],
        out_specs=(pl.BlockSpec((blk, D_OUT), lambda i: (i, 0)),
                   pl.BlockSpec((blk, D_OUT), lambda i: (i, 0))),
        out_shape=(jax.ShapeDtypeStruct((N_NODES, D_OUT), jnp.float32),
                   jax.ShapeDtypeStruct((N_NODES, D_OUT), jnp.float32)),
    )(t0, t1, g_col)


def kernel(x, edge_index, W1, b1, W2, b2):
    src = edge_index[0].astype(jnp.int32)
    dst = edge_index[1].astype(jnp.int32)
    n_pad = NCH * CHUNK - NE
    pad_idx = N_NODES + (jnp.arange(n_pad, dtype=jnp.int32) % (NP - N_NODES))
    src_flat = jnp.concatenate([src, pad_idx])
    dst_flat = jnp.concatenate([dst, pad_idx])
    src2d = src_flat.reshape(NCH, CHUNK)
    dst2d = dst_flat.reshape(NCH, CHUNK)

    w2p = jnp.pad(W2, ((0, 0), (0, D3 - D_OUT)))
    b1r = b1.reshape(1, D)
    b2r = jnp.pad(b2, (0, D3 - D_OUT)).reshape(1, D3)
    zeros_d = jnp.zeros((NP, D), jnp.float32)
    zeros_d3 = jnp.zeros((NP, D3), jnp.float32)
    zeros_16 = jnp.zeros((NP, 16), jnp.float32)
    ones_rows = jnp.ones((CHUNK, 16), jnp.float32)

    degp = _sc_degrees(src2d, dst2d, ones_rows, zeros_16)
    f_col, g_col = _tc_fg(degp)

    xs = _tc_scale(x, f_col)
    q = _sc_prop(xs, src2d, dst2d, zeros_d, D)
    h1s = _tc_linear(q[0], q[1], g_col, f_col, W1, b1r, relu=True)
    r = _sc_prop(h1s, src2d, dst2d, zeros_d, D)
    h2s = _tc_linear(r[0], r[1], g_col, f_col, w2p, b2r, relu=False)
    t = _sc_prop(h2s, src2d, dst2d, zeros_d3, D3)
    ls, h = _tc_final(t[0], t[1], g_col)
    return (ls, h)


# speedup vs baseline: 19.0745x; 1.0425x over previous
import functools

import jax
import jax.numpy as jnp
from jax import lax
from jax.experimental import pallas as pl
from jax.experimental.pallas import tpu as pltpu
from jax.experimental.pallas import tpu_sc as plsc

N_NODES = 10000
NP = 10240
D = 128
D3 = 64
D_OUT = 40
NE = 320000
CHUNK = 128
NC, NS = 2, 16
NW = NC * NS
NCH = 2560
CPW = NCH // NW
EPW = CPW * CHUNK
ROWS_PW = NP // NS
HBINS = NP // 16

_MESH = dict(core_axis_name="c", subcore_axis_name="s", num_cores=NC,
             num_subcores=NS)


def _wid():
    return lax.axis_index("s") * NC + lax.axis_index("c")


def _deg_body(src_hbm, dst_hbm, ones_hbm, zero_hbm, out_hbm,
              sidx_v, didx_v, ones_v, accs_sh, accd_sh):
    c = lax.axis_index("c")
    s = lax.axis_index("s")
    w = s * NC + c
    pltpu.sync_copy(src_hbm.at[pl.ds(w * CPW, CPW)], sidx_v)
    pltpu.sync_copy(dst_hbm.at[pl.ds(w * CPW, CPW)], didx_v)
    pltpu.sync_copy(ones_hbm, ones_v)
    r0 = s * ROWS_PW
    pltpu.sync_copy(zero_hbm.at[pl.ds(r0, ROWS_PW)], accs_sh.at[pl.ds(r0, ROWS_PW)])
    pltpu.sync_copy(zero_hbm.at[pl.ds(r0, ROWS_PW)], accd_sh.at[pl.ds(r0, ROWS_PW)])
    plsc.subcore_barrier()

    def step(j, _):
        pltpu.sync_copy(ones_v, accs_sh.at[sidx_v.at[j]], add=True)
        pltpu.sync_copy(ones_v, accd_sh.at[didx_v.at[j]], add=True)
        return _

    lax.fori_loop(0, CPW, step, 0)
    plsc.subcore_barrier()
    pltpu.sync_copy(accs_sh.at[pl.ds(r0, ROWS_PW)], out_hbm.at[c, 0, pl.ds(r0, ROWS_PW)])
    pltpu.sync_copy(accd_sh.at[pl.ds(r0, ROWS_PW)], out_hbm.at[c, 1, pl.ds(r0, ROWS_PW)])


def _sc_degrees(src2d, dst2d, ones_rows, zero_rows):
    k = pl.kernel(
        _deg_body,
        out_type=jax.ShapeDtypeStruct((NC, 2, NP, 16), jnp.float32),
        mesh=plsc.VectorSubcoreMesh(**_MESH),
        compiler_params=pltpu.CompilerParams(use_tc_tiling_on_sc=False),
        scratch_types=[
            pltpu.VMEM((CPW, CHUNK), jnp.int32),
            pltpu.VMEM((CPW, CHUNK), jnp.int32),
            pltpu.VMEM((CHUNK, 16), jnp.float32),
            pltpu.VMEM_SHARED((NP, 16), jnp.float32),
            pltpu.VMEM_SHARED((NP, 16), jnp.float32),
        ],
    )
    return k(src2d, dst2d, ones_rows, zero_rows)


NBUF = 2


HCH = CPW // 2


def _prop_body(table_hbm, src_hbm, dst_hbm, zero_hbm, out_hbm,
               sidx_v, didx_v, buf0_v, buf1_v, acc_sh, gsem0, gsem1):
    c = lax.axis_index("c")
    s = lax.axis_index("s")
    w = s * NC + c
    bufs = (buf0_v, buf1_v)
    gsems = (gsem0, gsem1)
    r0 = s * ROWS_PW
    pltpu.sync_copy(zero_hbm.at[pl.ds(r0, ROWS_PW)], acc_sh.at[pl.ds(r0, ROWS_PW)])
    plsc.subcore_barrier()

    for h in range(2):
        base = w * CPW + h * HCH
        pltpu.sync_copy(src_hbm.at[pl.ds(base, HCH)], sidx_v)
        pltpu.sync_copy(dst_hbm.at[pl.ds(base, HCH)], didx_v)
        pltpu.async_copy(table_hbm.at[sidx_v.at[0]], bufs[0], gsems[0])

        def group(g, _):
            for b in range(NBUF):
                j = g * NBUF + b
                pltpu.make_async_copy(
                    table_hbm.at[sidx_v.at[j]], bufs[b], gsems[b]).wait()

                @pl.when(j + 1 < HCH)
                def _issue():
                    pltpu.async_copy(table_hbm.at[sidx_v.at[j + 1]],
                                     bufs[1 - b], gsems[1 - b])

                pltpu.sync_copy(bufs[b], acc_sh.at[didx_v.at[j]], add=True)
            return _

        lax.fori_loop(0, HCH // NBUF, group, 0)
    plsc.subcore_barrier()
    pltpu.sync_copy(acc_sh.at[pl.ds(r0, ROWS_PW)], out_hbm.at[c, pl.ds(r0, ROWS_PW)])


def _sc_prop(table, src2d, dst2d, zero_rows, d):
    params = (pltpu.CompilerParams(use_tc_tiling_on_sc=False)
              if d < 128 else None)
    k = pl.kernel(
        _prop_body,
        out_type=jax.ShapeDtypeStruct((NC, NP, d), jnp.float32),
        mesh=plsc.VectorSubcoreMesh(**_MESH),
        compiler_params=params,
        scratch_types=[
            pltpu.VMEM((HCH, CHUNK), jnp.int32),
            pltpu.VMEM((HCH, CHUNK), jnp.int32),
            pltpu.VMEM((CHUNK, d), jnp.float32),
            pltpu.VMEM((CHUNK, d), jnp.float32),
            pltpu.VMEM_SHARED((NP, d), jnp.float32),
            pltpu.SemaphoreType.DMA,
            pltpu.SemaphoreType.DMA,
        ],
    )
    return k(table, src2d, dst2d, zero_rows)


def _fg_body(degp_ref, f_ref, g_ref):
    deg_o = degp_ref[0, 0, :, 0:1] + degp_ref[1, 0, :, 0:1]
    deg_i = degp_ref[0, 1, :, 0:1] + degp_ref[1, 1, :, 0:1]
    valid = lax.broadcasted_iota(jnp.int32, (NP, 1), 0) < N_NODES
    f_ref[...] = jnp.where(valid, lax.rsqrt(jnp.maximum(deg_o, 1.0)), 0.0)
    g_ref[...] = jnp.where(valid, lax.rsqrt(jnp.maximum(deg_i, 1.0)), 0.0)


def _tc_fg(degp):
    return pl.pallas_call(
        _fg_body,
        out_shape=(jax.ShapeDtypeStruct((NP, 1), jnp.float32),
                   jax.ShapeDtypeStruct((NP, 1), jnp.float32)),
    )(degp)


def _scale_body(x_ref, f_ref, o_ref):
    o_ref[...] = x_ref[...] * f_ref[...]


def _tc_scale(x, f_col):
    d = x.shape[1]
    blk = 2048
    return pl.pallas_call(
        _scale_body,
        grid=(NP // blk,),
        in_specs=[pl.BlockSpec((blk, d), lambda i: (i, 0)),
                  pl.BlockSpec((blk, 1), lambda i: (i, 0))],
        out_specs=pl.BlockSpec((blk, d), lambda i: (i, 0)),
        out_shape=jax.ShapeDtypeStruct((NP, d), jnp.float32),
    )(x, f_col)


def _lin_body(relu, q_ref, g_ref, f_ref, w_ref, b_ref, o_ref):
    q = (q_ref[0] + q_ref[1]) * g_ref[...]
    h = jnp.dot(q, w_ref[...], preferred_element_type=jnp.float32) + b_ref[...]
    if relu:
        h = jnp.maximum(h, 0.0)
    o_ref[...] = h * f_ref[...]


def _tc_linear(q, g_col, f_col, w, b_row, relu):
    _, n, d_in = q.shape
    d_out = w.shape[1]
    blk = 2048
    return pl.pallas_call(
        functools.partial(_lin_body, relu),
        grid=(n // blk,),
        in_specs=[pl.BlockSpec((NC, blk, d_in), lambda i: (0, i, 0)),
                  pl.BlockSpec((blk, 1), lambda i: (i, 0)),
                  pl.BlockSpec((blk, 1), lambda i: (i, 0)),
                  pl.BlockSpec((d_in, d_out), lambda i: (0, 0)),
                  pl.BlockSpec((1, d_out), lambda i: (0, 0))],
        out_specs=pl.BlockSpec((blk, d_out), lambda i: (i, 0)),
        out_shape=jax.ShapeDtypeStruct((n, d_out), jnp.float32),
    )(q, g_col, f_col, w, b_row)


def _final_body(t_ref, g_ref, ls_ref, h_ref):
    h = ((t_ref[0] + t_ref[1]) * g_ref[...])[:, :D_OUT]
    m = jnp.max(h, axis=1, keepdims=True)
    e = jnp.exp(h - m)
    lse = jnp.log(jnp.sum(e, axis=1, keepdims=True)) + m
    h_ref[...] = h
    ls_ref[...] = h - lse


def _tc_final(t, g_col):
    d = t.shape[2]
    blk = 1000
    return pl.pallas_call(
        _final_body,
        grid=(N_NODES // blk,),
        in_specs=[pl.BlockSpec((NC, blk, d), lambda i: (0, i, 0)),
                  pl.BlockSpec((blk, 1), lambda i: (i, 0))],
        out_specs=(pl.BlockSpec((blk, D_OUT), lambda i: (i, 0)),
                   pl.BlockSpec((blk, D_OUT), lambda i: (i, 0))),
        out_shape=(jax.ShapeDtypeStruct((N_NODES, D_OUT), jnp.float32),
                   jax.ShapeDtypeStruct((N_NODES, D_OUT), jnp.float32)),
    )(t, g_col)


def kernel(x, edge_index, W1, b1, W2, b2):
    src = edge_index[0].astype(jnp.int32)
    dst = edge_index[1].astype(jnp.int32)
    n_pad = NCH * CHUNK - NE
    pad_idx = N_NODES + (jnp.arange(n_pad, dtype=jnp.int32) % (NP - N_NODES))
    src_flat = jnp.concatenate([src, pad_idx])
    dst_flat = jnp.concatenate([dst, pad_idx])
    src2d = src_flat.reshape(NCH, CHUNK)
    dst2d = dst_flat.reshape(NCH, CHUNK)

    w2p = jnp.pad(W2, ((0, 0), (0, D3 - D_OUT)))
    b1r = b1.reshape(1, D)
    b2r = jnp.pad(b2, (0, D3 - D_OUT)).reshape(1, D3)
    zeros_d = jnp.zeros((NP, D), jnp.float32)
    zeros_d3 = jnp.zeros((NP, D3), jnp.float32)
    zeros_16 = jnp.zeros((NP, 16), jnp.float32)
    ones_rows = jnp.ones((CHUNK, 16), jnp.float32)

    degp = _sc_degrees(src2d, dst2d, ones_rows, zeros_16)
    f_col, g_col = _tc_fg(degp)

    xs = _tc_scale(x, f_col)
    q = _sc_prop(xs, src2d, dst2d, zeros_d, D)
    h1s = _tc_linear(q, g_col, f_col, W1, b1r, relu=True)
    r = _sc_prop(h1s, src2d, dst2d, zeros_d, D)
    h2s = _tc_linear(r, g_col, f_col, w2p, b2r, relu=False)
    t = _sc_prop(h2s, src2d, dst2d, zeros_d3, D3)
    ls, h = _tc_final(t, g_col)
    return (ls, h)
